# Initial kernel scaffold; baseline (speedup 1.0000x reference)
import jax
import jax.numpy as jnp
from jax.experimental import pallas as pl


def kernel(adj, feat, eps):
    raise NotImplementedError("write your pallas kernel here")


# speedup vs baseline: 1.0329x; 1.0329x over previous
import jax
import jax.numpy as jnp
from jax.experimental import pallas as pl
from jax.experimental.pallas import tpu as pltpu

_BM = 400


def _gin_block(adj_ref, feat_ref, eps_ref, out_ref):
    i = pl.program_id(0)
    bm = out_ref.shape[0]
    neigh = jnp.dot(adj_ref[...], feat_ref[...],
                    preferred_element_type=jnp.float32)
    scale = 1.0 + eps_ref[0, 0]
    out_ref[...] = neigh + scale * feat_ref[pl.ds(i * bm, bm), :]


def kernel(adj, feat, eps):
    n, d = feat.shape
    bm = _BM
    eps2 = eps.reshape(1, 1)
    return pl.pallas_call(
        _gin_block,
        grid=(n // bm,),
        in_specs=[
            pl.BlockSpec((bm, n), lambda i: (i, 0)),
            pl.BlockSpec((n, d), lambda i: (0, 0)),
            pl.BlockSpec(memory_space=pltpu.SMEM),
        ],
        out_specs=pl.BlockSpec((bm, d), lambda i: (i, 0)),
        out_shape=jax.ShapeDtypeStruct((n, d), jnp.float32),
        compiler_params=pltpu.CompilerParams(
            dimension_semantics=("arbitrary",),
        ),
    )(adj, feat, eps2)
